# baseline (speedup 1.0000x reference)
import functools

import jax
import jax.numpy as jnp
from jax import lax
from jax.experimental import pallas as pl
from jax.experimental.pallas import tpu as pltpu
from jax.experimental.pallas import tpu_sc as plsc

B = 16384
N_NUM = 13
N_CAT = 26
VOCAB = 100001
EMB = 50

_INFO = plsc.get_sparse_core_info()
NC = _INFO.num_cores
NS = _INFO.num_subcores
NW = NC * NS

T8_ROWS = (N_CAT * VOCAB * EMB) // 8
OUT_PAD = 1320


def _probe(table8):
    mesh = plsc.VectorSubcoreMesh(core_axis_name="c", subcore_axis_name="s")

    @functools.partial(
        pl.kernel,
        mesh=mesh,
        out_type=jax.ShapeDtypeStruct((B, OUT_PAD), jnp.float32),
        scratch_types=[
            pltpu.VMEM((64, OUT_PAD), jnp.float32),
            pltpu.VMEM((8, 8), jnp.float32),
            pltpu.SemaphoreType.DMA,
        ],
        compiler_params=pltpu.CompilerParams(use_tc_tiling_on_sc=False),
    )
    def k(table_hbm, out_hbm, buf, tbuf, sem):
        wid = lax.axis_index("s") * NC + lax.axis_index("c")
        rows_per_w = B // NW

        def body(ci, _):
            r0 = wid * rows_per_w + ci * 64
            pltpu.sync_copy(table_hbm.at[pl.ds(0, 8)], tbuf)
            pltpu.sync_copy(buf, out_hbm.at[pl.ds(r0, 64)])
            return 0

        lax.fori_loop(0, rows_per_w // 64, body, 0)

    return k(table8)


def kernel(x_num, x_cat, tables):
    table8 = tables.reshape(-1)[: T8_ROWS * 8].reshape(T8_ROWS, 8)
    out = _probe(table8)
    return out[:, : N_NUM + N_CAT * EMB]


# speedup vs baseline: 1.8360x; 1.8360x over previous
import functools

import jax
import jax.numpy as jnp
from jax import lax
from jax.experimental import pallas as pl
from jax.experimental import layout as jex_layout
from jax.experimental.pallas import tpu as pltpu
from jax.experimental.pallas import tpu_sc as plsc

B = 16384
N_NUM = 13
N_CAT = 26
VOCAB = 100001
EMB = 50
EMB_PAD = 56
OUT_W = N_NUM + N_CAT * EMB
OUT_PAD = 1320

_INFO = plsc.get_sparse_core_info()
NC = _INFO.num_cores
NS = _INFO.num_subcores
NW = NC * NS

CB = 32
N_CHUNK_TOT = B // CB
N_CHUNKS = N_CHUNK_TOT // NW
ROWS_PER_CHUNK = CB * N_CAT
G = 13
GW = ROWS_PER_CHUNK // G


def _project(idx3, x3, padded_tables):
    mesh = plsc.VectorSubcoreMesh(core_axis_name="c", subcore_axis_name="s")

    @functools.partial(
        pl.kernel,
        mesh=mesh,
        out_type=jax.ShapeDtypeStruct((B, OUT_PAD), jnp.float32),
        scratch_types=[
            pltpu.VMEM((G, GW), jnp.int32),
            pltpu.VMEM((CB, 16), jnp.float32),
            pltpu.VMEM((ROWS_PER_CHUNK, EMB_PAD), jnp.float32),
            pltpu.VMEM((CB, OUT_PAD), jnp.float32),
            pltpu.SemaphoreType.DMA,
        ],
        compiler_params=pltpu.CompilerParams(
            use_tc_tiling_on_sc=False, needs_layout_passes=False
        ),
    )
    def k(idx_hbm, x_hbm, table_hbm, out_hbm, idx_v, x_v, rows_v, outbuf, sem):
        wid = lax.axis_index("s") * NC + lax.axis_index("c")

        def body(ci, _):
            c = wid * N_CHUNKS + ci
            pltpu.sync_copy(idx_hbm.at[c], idx_v)
            pltpu.sync_copy(x_hbm.at[c], x_v)
            copies = [
                pltpu.async_copy(
                    table_hbm.at[idx_v.at[j]],
                    rows_v.at[pl.ds(j * GW, GW)],
                    sem,
                )
                for j in range(G)
            ]
            for cp in copies:
                cp.wait()

            def row_body(m, _):
                outbuf[m, pl.ds(0, 16)] = x_v[m]
                brow0 = m * N_CAT
                for f in range(N_CAT):
                    src = rows_v.at[brow0 + f]
                    d0 = N_NUM + EMB * f
                    outbuf[m, pl.ds(d0, 16)] = src[pl.ds(0, 16)]
                    outbuf[m, pl.ds(d0 + 16, 16)] = src[pl.ds(16, 16)]
                    outbuf[m, pl.ds(d0 + 32, 16)] = src[pl.ds(32, 16)]
                    outbuf[m, pl.ds(d0 + 40, 16)] = src[pl.ds(40, 16)]
                return 0

            lax.fori_loop(0, CB, row_body, 0)
            pltpu.sync_copy(outbuf, out_hbm.at[pl.ds(c * CB, CB)])
            return 0

        lax.fori_loop(0, N_CHUNKS, body, 0)

    return k(idx3, x3, padded_tables)


def kernel(x_num, x_cat, tables):
    flat_tables = tables.reshape(N_CAT * VOCAB, EMB)
    padded_tables = jnp.pad(flat_tables, ((0, 0), (0, EMB_PAD - EMB)))
    try:
        fmt = jex_layout.Format(
            jex_layout.Layout(major_to_minor=(0, 1), tiling=((8,),))
        )
        padded_tables = jex_layout.with_layout_constraint(padded_tables, fmt)
    except Exception:
        pass
    idx = x_cat + jnp.arange(N_CAT, dtype=jnp.int32) * VOCAB
    idx3 = idx.reshape(N_CHUNK_TOT, G, GW)
    x3 = jnp.pad(x_num, ((0, 0), (0, 3))).reshape(N_CHUNK_TOT, CB, 16)
    out = _project(idx3, x3, padded_tables)
    return out[:, :OUT_W]
